# baseline (speedup 1.0000x reference)
import functools

import jax
import jax.numpy as jnp
from jax import lax
from jax.experimental import pallas as pl
from jax.experimental.pallas import tpu as pltpu
from jax.experimental.pallas import tpu_sc as plsc

NUM_CORES = 2
NUM_SUBCORES = 16
NW = NUM_CORES * NUM_SUBCORES
CHUNK = 128
NBUF = 4


@functools.lru_cache(maxsize=None)
def _build_gather(n_rows: int, d_model: int):
    assert n_rows % (NW * CHUNK) == 0
    b_per_w = n_rows // NW
    n_steps = b_per_w // CHUNK
    mesh = plsc.VectorSubcoreMesh(
        core_axis_name="c", subcore_axis_name="s", num_cores=NUM_CORES
    )

    @functools.partial(
        pl.kernel,
        out_type=jax.ShapeDtypeStruct((n_rows, d_model), jnp.float32),
        mesh=mesh,
        scratch_types=(
            [pltpu.VMEM((b_per_w,), jnp.int32)]
            + [pltpu.VMEM((CHUNK, d_model), jnp.float32) for _ in range(NBUF)]
            + [pltpu.SemaphoreType.DMA for _ in range(2 * NBUF)]
        ),
    )
    def gather_kernel(idx_hbm, table_hbm, out_hbm, idx_v, *bufs_and_sems):
        rows = bufs_and_sems[:NBUF]
        gsem = bufs_and_sems[NBUF : 2 * NBUF]
        wsem = bufs_and_sems[2 * NBUF :]
        wid = lax.axis_index("s") * NUM_CORES + lax.axis_index("c")
        base = wid * b_per_w
        pltpu.sync_copy(idx_hbm.at[pl.ds(base, b_per_w)], idx_v)

        def gather_src(j):
            return table_hbm.at[idx_v.at[pl.ds(j * CHUNK, CHUNK)]]

        def out_dst(j):
            return out_hbm.at[pl.ds(base + j * CHUNK, CHUNK)]

        for b in range(NBUF):
            pltpu.async_copy(gather_src(b), rows[b], gsem[b])

        n_outer = n_steps // NBUF

        @pl.loop(0, n_outer)
        def _outer(p):
            j0 = p * NBUF
            for b in range(NBUF):
                pltpu.make_async_copy(gather_src(j0 + b), rows[b], gsem[b]).wait()
                pltpu.async_copy(rows[b], out_dst(j0 + b), wsem[b])

            @pl.when(p < n_outer - 1)
            def _refill():
                for b in range(NBUF):
                    pltpu.make_async_copy(rows[b], out_dst(j0 + b), wsem[b]).wait()
                    pltpu.async_copy(gather_src(j0 + NBUF + b), rows[b], gsem[b])

        for b in range(NBUF):
            pltpu.make_async_copy(rows[b], out_dst(n_steps - NBUF + b), wsem[b]).wait()

    return gather_kernel


def kernel(sequencee, table):
    b, s = sequencee.shape
    v, d = table.shape
    flat_idx = sequencee.reshape(b * s).astype(jnp.int32)
    out = _build_gather(b * s, d)(flat_idx, table)
    return out.reshape(b, s, d)
